# baseline (speedup 1.0000x reference)
import functools

import jax
import jax.numpy as jnp
from jax import lax
from jax.experimental import pallas as pl
from jax.experimental.pallas import tpu as pltpu
from jax.experimental.pallas import tpu_sc as plsc

B_SEQ = 4096
T_IN = 200
T_OUT = T_IN + 1
D = 128
IDS_LEN = B_SEQ * T_IN
TOTAL_ROWS = B_SEQ * T_OUT
N_WORKERS = 32
PER_WORKER = TOTAL_ROWS // N_WORKERS
CHUNK = 192
N_CHUNKS = PER_WORKER // CHUNK
DEPTH = 4
QUADS = 32
BOS_OFF = DEPTH * CHUNK
SBASE = (B_SEQ // CHUNK) * CHUNK
SHIFT = B_SEQ - SBASE

_MESH = plsc.VectorSubcoreMesh(core_axis_name="c", subcore_axis_name="s")


@functools.partial(
    pl.kernel,
    mesh=_MESH,
    out_type=jax.ShapeDtypeStruct((TOTAL_ROWS, D), jnp.float32),
    scratch_types=[
        pltpu.VMEM((CHUNK,), jnp.int32),
        pltpu.VMEM((CHUNK,), jnp.int32),
        pltpu.VMEM((CHUNK,), jnp.int32),
        pltpu.VMEM((CHUNK,), jnp.int32),
        pltpu.VMEM(((DEPTH + 1) * CHUNK, D), jnp.float32),
        pltpu.SemaphoreType.DMA,
        pltpu.SemaphoreType.DMA,
        pltpu.SemaphoreType.DMA,
        pltpu.SemaphoreType.DMA,
        pltpu.SemaphoreType.DMA,
        pltpu.SemaphoreType.DMA,
        pltpu.SemaphoreType.DMA,
        pltpu.SemaphoreType.DMA,
    ],
)
def _sc_embed(ids_hbm, table_hbm, out_hbm,
              ix0, ix1, ix2, ix3, rb,
              sg0, sg1, sg2, sg3, ss0, ss1, ss2, ss3):
    ixs = (ix0, ix1, ix2, ix3)
    sgs, sss = (sg0, sg1, sg2, sg3), (ss0, ss1, ss2, ss3)

    wid = lax.axis_index("s") * 2 + lax.axis_index("c")
    base0 = wid * PER_WORKER
    zero = jnp.full((16,), 0, jnp.int32)

    for g in range(CHUNK // 16):
        ix0[pl.ds(16 * g, 16)] = zero
    pltpu.async_copy(table_hbm.at[ix0], rb.at[pl.ds(BOS_OFF, CHUNK)], sg0)
    pltpu.make_async_copy(table_hbm.at[ix0],
                          rb.at[pl.ds(BOS_OFF, CHUNK)], sg0).wait()

    def build_idx(base, b):
        al = jnp.maximum(base - B_SEQ, jnp.int32(0))
        al = pl.multiple_of(al, 8)
        pltpu.sync_copy(ids_hbm.at[pl.ds(al, CHUNK)], ixs[b])
        straddle = base == SBASE
        for k in reversed(range(CHUNK // 16)):
            v = ixs[b][pl.ds(16 * k, 16)]
            if 16 * k >= SHIFT:
                alt = ixs[b][pl.ds(16 * k - SHIFT, 16)]
            else:
                alt = zero
            ixs[b][pl.ds(16 * k, 16)] = jnp.where(straddle, alt, v)

    def issue_g(b):
        pltpu.async_copy(table_hbm.at[ixs[b]], rb.at[pl.ds(b * CHUNK, CHUNK)],
                         sgs[b])

    def wait_g(b):
        pltpu.make_async_copy(table_hbm.at[ixs[b]],
                              rb.at[pl.ds(b * CHUNK, CHUNK)], sgs[b]).wait()

    def body(i, b, more):
        base = base0 + i * CHUNK
        wait_g(b)
        off = jnp.where(base + CHUNK <= B_SEQ,
                        jnp.int32(BOS_OFF), jnp.int32(b * CHUNK))
        off = pl.multiple_of(off, 8)
        store = pltpu.async_copy(
            rb.at[pl.ds(off, CHUNK)],
            out_hbm.at[pl.ds(base, CHUNK)], sss[b])
        if more:
            build_idx(base0 + (i + DEPTH) * CHUNK, b)
        store.wait()
        if more:
            issue_g(b)

    for b in range(DEPTH):
        build_idx(base0 + b * CHUNK, b)
        issue_g(b)

    def quad(p, carry):
        for b in range(DEPTH):
            body(DEPTH * p + b, b, True)
        return carry

    lax.fori_loop(0, QUADS, quad, jnp.int32(0))
    body(jnp.int32(N_CHUNKS - 6), 0, True)
    body(jnp.int32(N_CHUNKS - 5), 1, True)
    body(jnp.int32(N_CHUNKS - 4), 2, False)
    body(jnp.int32(N_CHUNKS - 3), 3, False)
    body(jnp.int32(N_CHUNKS - 2), 0, False)
    body(jnp.int32(N_CHUNKS - 1), 1, False)


def kernel(input_ids, table):
    ids_t = jnp.transpose(input_ids.astype(jnp.int32)).reshape(IDS_LEN)
    out = _sc_embed(ids_t, table)
    return out.reshape(T_OUT, B_SEQ, D).transpose(1, 0, 2)


# speedup vs baseline: 1.1725x; 1.1725x over previous
import functools

import jax
import jax.numpy as jnp
from jax import lax
from jax.experimental import pallas as pl
from jax.experimental.pallas import tpu as pltpu
from jax.experimental.pallas import tpu_sc as plsc

B_SEQ = 4096
T_IN = 200
T_OUT = T_IN + 1
D = 128
IDS_LEN = B_SEQ * T_IN
TOTAL_ROWS = B_SEQ * T_OUT
N_WORKERS = 32
PER_WORKER = TOTAL_ROWS // N_WORKERS
CHUNK = 128
N_CHUNKS = PER_WORKER // CHUNK
DEPTH = 4
QUADS = 49
BOS_OFF = DEPTH * CHUNK

_MESH = plsc.VectorSubcoreMesh(core_axis_name="c", subcore_axis_name="s")


@functools.partial(
    pl.kernel,
    mesh=_MESH,
    out_type=jax.ShapeDtypeStruct((TOTAL_ROWS, D), jnp.float32),
    scratch_types=[
        pltpu.VMEM((CHUNK,), jnp.int32),
        pltpu.VMEM((CHUNK,), jnp.int32),
        pltpu.VMEM((CHUNK,), jnp.int32),
        pltpu.VMEM((CHUNK,), jnp.int32),
        pltpu.VMEM(((DEPTH + 1) * CHUNK, D), jnp.float32),
        pltpu.SemaphoreType.DMA,
        pltpu.SemaphoreType.DMA,
        pltpu.SemaphoreType.DMA,
        pltpu.SemaphoreType.DMA,
        pltpu.SemaphoreType.DMA,
        pltpu.SemaphoreType.DMA,
        pltpu.SemaphoreType.DMA,
        pltpu.SemaphoreType.DMA,
    ],
)
def _sc_embed(ids_hbm, table_hbm, out_hbm,
              ix0, ix1, ix2, ix3, rb,
              sg0, sg1, sg2, sg3, ss0, ss1, ss2, ss3):
    ixs = (ix0, ix1, ix2, ix3)
    sgs, sss = (sg0, sg1, sg2, sg3), (ss0, ss1, ss2, ss3)

    wid = lax.axis_index("s") * 2 + lax.axis_index("c")
    base0 = wid * PER_WORKER
    zero = jnp.full((16,), 0, jnp.int32)

    for g in range(CHUNK // 16):
        ix0[pl.ds(16 * g, 16)] = zero
    pltpu.async_copy(table_hbm.at[ix0], rb.at[pl.ds(BOS_OFF, CHUNK)], sg0)
    pltpu.make_async_copy(table_hbm.at[ix0],
                          rb.at[pl.ds(BOS_OFF, CHUNK)], sg0).wait()

    def build_idx(base, b):
        al = jnp.maximum(base - B_SEQ, jnp.int32(0))
        al = pl.multiple_of(al, 8)
        pltpu.sync_copy(ids_hbm.at[pl.ds(al, CHUNK)], ixs[b])

    def issue_g(b):
        pltpu.async_copy(table_hbm.at[ixs[b]], rb.at[pl.ds(b * CHUNK, CHUNK)],
                         sgs[b])

    def wait_g(b):
        pltpu.make_async_copy(table_hbm.at[ixs[b]],
                              rb.at[pl.ds(b * CHUNK, CHUNK)], sgs[b]).wait()

    def body(i, b, more):
        base = base0 + i * CHUNK
        wait_g(b)
        off = jnp.where(base < B_SEQ, jnp.int32(BOS_OFF), jnp.int32(b * CHUNK))
        off = pl.multiple_of(off, 8)
        store = pltpu.async_copy(
            rb.at[pl.ds(off, CHUNK)],
            out_hbm.at[pl.ds(base, CHUNK)], sss[b])
        if more:
            build_idx(base0 + (i + DEPTH) * CHUNK, b)
        store.wait()
        if more:
            issue_g(b)

    for b in range(DEPTH):
        build_idx(base0 + b * CHUNK, b)
        issue_g(b)

    def quad(p, carry):
        for b in range(DEPTH):
            body(DEPTH * p + b, b, True)
        return carry

    lax.fori_loop(0, QUADS, quad, jnp.int32(0))
    body(jnp.int32(N_CHUNKS - 5), 0, True)
    body(jnp.int32(N_CHUNKS - 4), 1, False)
    body(jnp.int32(N_CHUNKS - 3), 2, False)
    body(jnp.int32(N_CHUNKS - 2), 3, False)
    body(jnp.int32(N_CHUNKS - 1), 0, False)


def kernel(input_ids, table):
    ids_t = jnp.transpose(input_ids.astype(jnp.int32)).reshape(IDS_LEN)
    out = _sc_embed(ids_t, table)
    return out.reshape(T_OUT, B_SEQ, D).transpose(1, 0, 2)
